# Initial kernel scaffold; baseline (speedup 1.0000x reference)
import jax
import jax.numpy as jnp
from jax.experimental import pallas as pl


def kernel(output, labels, images, reconstructions):
    raise NotImplementedError("write your pallas kernel here")


# speedup vs baseline: 1.0434x; 1.0434x over previous
import functools

import jax
import jax.numpy as jnp
from jax import lax
from jax.experimental import pallas as pl
from jax.experimental.pallas import tpu as pltpu
from jax.experimental.pallas import tpu_sc as plsc

_GAMMA = 2.0
_ALPHA = 0.5
_NUM_HARD = 2
_RECON_SCALE = 1e-06

_NC = 2
_NS = 16
_NW = _NC * _NS


def _sigmoid(x):
    return jnp.where(
        x >= 0.0,
        1.0 / (1.0 + jnp.exp(-x)),
        jnp.exp(x) / (1.0 + jnp.exp(x)),
    )


def _sc_topk(out_flat, lab_flat):
    flat = out_flat.shape[0]
    per_tile = flat // _NW
    groups = per_tile // 80

    mesh = plsc.VectorSubcoreMesh(
        core_axis_name="c", subcore_axis_name="s",
        num_cores=_NC, num_subcores=_NS)

    @functools.partial(
        pl.kernel,
        out_type=jax.ShapeDtypeStruct((_NW, 16), jnp.float32),
        mesh=mesh,
        compiler_params=pltpu.CompilerParams(needs_layout_passes=False),
        scratch_types=[
            pltpu.VMEM((per_tile,), jnp.float32),
            pltpu.VMEM((per_tile,), jnp.float32),
            pltpu.VMEM((16,), jnp.float32),
        ],
    )
    def topk_kernel(out_hbm, lab_hbm, cand_hbm, o_v, l_v, c_v):
        wid = lax.axis_index("s") * _NC + lax.axis_index("c")
        base = wid * per_tile
        pltpu.sync_copy(out_hbm.at[pl.ds(base, per_tile)], o_v)
        pltpu.sync_copy(lab_hbm.at[pl.ds(base, per_tile)], l_v)

        lane = lax.iota(jnp.int32, 16)
        zero_idx = jnp.zeros((16,), jnp.int32)
        cmasks = [((lane + v * 16) % 5) == 0 for v in range(5)]

        def merge(cand, z):
            zs = lax.sort(z)
            zrev = lax.rev(zs, (0,))
            return lax.sort(jnp.maximum(cand, zrev))

        def body(g, carry):
            cand, thresh = carry
            b = g * 80
            zs = []
            zmax = jnp.full((16,), -jnp.inf, jnp.float32)
            for v in range(5):
                o = o_v[pl.ds(b + v * 16, 16)]
                l = l_v[pl.ds(b + v * 16, 16)]
                m = (l < -0.5) & cmasks[v]
                z = jnp.where(m, o, -jnp.inf)
                zs.append(z)
                zmax = jnp.maximum(zmax, z)
            hit = jnp.any(zmax > thresh)

            def do_merge():
                c = cand
                for z in zs:
                    c = merge(c, z)
                return c, jnp.take(c, zero_idx)

            return lax.cond(hit, do_merge, lambda: (cand, thresh))

        init = (jnp.full((16,), -jnp.inf, jnp.float32),
                jnp.full((16,), -jnp.inf, jnp.float32))
        cand, _ = lax.fori_loop(0, groups, body, init)
        c_v[...] = cand
        pltpu.sync_copy(c_v, cand_hbm.at[wid])

    return topk_kernel(out_flat, lab_flat)


def _tc_main(o2, l2, im2, re2, grid):
    r = o2.shape[0] // grid
    rv = im2.shape[0] // grid

    def body(o_ref, l_ref, im_ref, re_ref, acc_ref):
        g = pl.program_id(0)

        @pl.when(g == 0)
        def _init():
            acc_ref[...] = jnp.zeros_like(acc_ref)

        s = o_ref[...]
        c = l_ref[...]
        ii = lax.broadcasted_iota(jnp.int32, (r, 128), 0)
        ll = lax.broadcasted_iota(jnp.int32, (r, 128), 1)
        col = (ii * 3 + ll) % 5
        c0 = col == 0
        pos0 = (c > 0.5) & c0
        neg0 = (c < -0.5) & c0
        p = _sigmoid(s)
        logp = jnp.log(p)
        focal = -((1.0 - p) ** 2) * ((1.0 - _ALPHA) * logp)
        focal = jnp.where(pos0, focal, 0.0)

        d = s - c
        ad = jnp.abs(d)
        l1 = jnp.where(ad < 1.0, 0.5 * d * d, ad - 0.5)

        rows = [jnp.sum(focal, axis=0)]
        for j in (1, 2, 3, 4):
            a = pltpu.roll(c, j, axis=1)
            b = pltpu.roll(a, 1, axis=0)
            cj = jnp.where(ll >= j, a, b)
            mj = (cj > 0.5) & (col == j)
            rows.append(jnp.sum(jnp.where(mj, l1, 0.0), axis=0))
        rows.append(jnp.sum(pos0.astype(jnp.float32), axis=0))
        rows.append(jnp.sum((pos0 & (p >= 0.5)).astype(jnp.float32), axis=0))
        rows.append(jnp.sum(neg0.astype(jnp.float32), axis=0))
        dr = re_ref[...] - im_ref[...]
        rows.append(jnp.sum(dr * dr, axis=0))
        upd = jnp.stack(rows, axis=0)
        upd = jnp.concatenate(
            [upd, jnp.zeros((16 - len(rows), 128), jnp.float32)], axis=0)
        acc_ref[...] += upd

    return pl.pallas_call(
        body,
        grid=(grid,),
        in_specs=[
            pl.BlockSpec((r, 128), lambda g: (g, 0)),
            pl.BlockSpec((r, 128), lambda g: (g, 0)),
            pl.BlockSpec((rv, 128), lambda g: (g, 0)),
            pl.BlockSpec((rv, 128), lambda g: (g, 0)),
        ],
        out_specs=pl.BlockSpec((16, 128), lambda g: (0, 0)),
        out_shape=jax.ShapeDtypeStruct((16, 128), jnp.float32),
    )(o2, l2, im2, re2)


def _tc_finalize(acc, cands, k, n_vox):
    def body(acc_ref, cand_ref, out_ref):
        a = acc_ref[...]
        f_pos = jnp.sum(a[0:1, :])
        s1 = jnp.sum(a[1:2, :])
        s2 = jnp.sum(a[2:3, :])
        s3 = jnp.sum(a[3:4, :])
        s4 = jnp.sum(a[4:5, :])
        pos_cnt = jnp.sum(a[5:6, :])
        pos_cor = jnp.sum(a[6:7, :])
        neg_cnt = jnp.sum(a[7:8, :])
        sse = jnp.sum(a[8:9, :])

        z = cand_ref[...]
        zr, zc = z.shape
        ii = lax.broadcasted_iota(jnp.int32, (zr, zc), 0)
        ll = lax.broadcasted_iota(jnp.int32, (zr, zc), 1)
        fpos = ii * zc + ll
        neg_f = jnp.float32(0.0)
        neg_c = jnp.float32(0.0)
        for _ in range(k):
            m = jnp.max(z)
            p = _sigmoid(m)
            pt = 1.0 - p
            contrib = -((1.0 - pt) ** 2) * (_ALPHA * jnp.log(pt))
            valid = m > -jnp.inf
            neg_f += jnp.where(valid, contrib, 0.0)
            neg_c += jnp.where(valid & (p < 0.5), 1.0, 0.0)
            first = jnp.min(jnp.where(z == m, fpos, jnp.int32(2**30)))
            z = jnp.where(fpos == first, -jnp.inf, z)

        neg_k = jnp.minimum(neg_cnt, jnp.float32(k))
        classify = (f_pos + neg_f) / (pos_cnt + neg_k)
        denom = jnp.maximum(pos_cnt, 1.0)
        rl1 = jnp.where(pos_cnt > 0, s1 / denom, 0.0)
        rl2 = jnp.where(pos_cnt > 0, s2 / denom, 0.0)
        rl3 = jnp.where(pos_cnt > 0, s3 / denom, 0.0)
        rl4 = jnp.where(pos_cnt > 0, s4 / denom, 0.0)
        recon = _RECON_SCALE * (sse / jnp.float32(n_vox))
        loss = classify + rl1 + rl2 + rl3 + rl4 + recon

        lo = lax.broadcasted_iota(jnp.int32, (1, 128), 1)
        vec = jnp.where(lo == 0, loss,
              jnp.where(lo == 1, classify,
              jnp.where(lo == 2, rl1,
              jnp.where(lo == 3, rl2,
              jnp.where(lo == 4, rl3,
              jnp.where(lo == 5, rl4,
              jnp.where(lo == 6, pos_cor,
              jnp.where(lo == 7, pos_cnt,
              jnp.where(lo == 8, neg_c,
              jnp.where(lo == 9, neg_k,
              jnp.where(lo == 10, recon, 0.0)))))))))))
        out_ref[...] = vec

    return pl.pallas_call(
        body,
        out_shape=jax.ShapeDtypeStruct((1, 128), jnp.float32),
    )(acc, cands)


def kernel(output, labels, images, reconstructions):
    n_rows = output.size // 5
    flat = output.size
    k = min(_NUM_HARD * output.shape[0], n_rows)

    of = output.reshape(-1)
    lf = labels.reshape(-1)
    o2 = of.reshape(-1, 128)
    l2 = lf.reshape(-1, 128)
    im2 = images.reshape(-1, 128)
    re2 = reconstructions.reshape(-1, 128)

    cands = _sc_topk(of, lf)
    acc = _tc_main(o2, l2, im2, re2, grid=27)
    res = _tc_finalize(acc, cands.reshape(4, 128), k, images.size)

    r = res[0]
    i32 = jnp.int32
    return (
        r[0], r[1], r[2], r[3], r[4], r[5],
        r[6].astype(i32), r[7].astype(i32),
        r[8].astype(i32), r[9].astype(i32),
        r[10],
    )


# speedup vs baseline: 36.3477x; 34.8360x over previous
import functools

import jax
import jax.numpy as jnp
from jax import lax
from jax.experimental import pallas as pl
from jax.experimental.pallas import tpu as pltpu
from jax.experimental.pallas import tpu_sc as plsc

_GAMMA = 2.0
_ALPHA = 0.5
_NUM_HARD = 2
_RECON_SCALE = 1e-06

_NC = 2
_NS = 16
_NW = _NC * _NS

_NEG_INF = float("-inf")


def _sigmoid(x):
    return jnp.where(
        x >= 0.0,
        1.0 / (1.0 + jnp.exp(-x)),
        jnp.exp(x) / (1.0 + jnp.exp(x)),
    )


def _tc_main(o3, l3, im2, re2, grid):
    ngroups = o3.shape[0]
    gb = ngroups // grid
    rv = im2.shape[0] // grid

    def body(o_ref, l_ref, im_ref, re_ref, acc_ref, z_ref):
        g = pl.program_id(0)

        @pl.when(g == 0)
        def _init():
            acc_ref[...] = jnp.zeros_like(acc_ref)

        s = o_ref[:, 0:24, :]
        c = l_ref[:, 0:24, :]
        pos = c > 0.5
        neg = c < -0.5
        p = _sigmoid(s)
        logp = jnp.log(p)
        focal = -((1.0 - p) ** 2) * ((1.0 - _ALPHA) * logp)
        focal = jnp.where(pos, focal, 0.0)

        def lanesum(x):
            v = jnp.sum(x.reshape(gb * 24, 24), axis=0, keepdims=True)
            return jnp.concatenate(
                [v, jnp.zeros((1, 104), jnp.float32)], axis=1)

        rows = [lanesum(focal)]
        for w in (1, 2, 3, 4):
            dw = o_ref[:, 24 * w:24 * w + 24, :] - l_ref[:, 24 * w:24 * w + 24, :]
            ad = jnp.abs(dw)
            l1 = jnp.where(ad < 1.0, 0.5 * dw * dw, ad - 0.5)
            rows.append(lanesum(jnp.where(pos, l1, 0.0)))
        rows.append(lanesum(pos.astype(jnp.float32)))
        rows.append(lanesum((pos & (p >= 0.5)).astype(jnp.float32)))
        rows.append(lanesum(neg.astype(jnp.float32)))

        dr = re_ref[...] - im_ref[...]
        v = jnp.sum(dr * dr, axis=0, keepdims=True)
        rows.append(jnp.concatenate(
            [v, jnp.zeros((1, 32), jnp.float32)], axis=1))

        upd = jnp.concatenate(rows, axis=0)
        upd = jnp.concatenate(
            [upd, jnp.zeros((16 - len(rows), 128), jnp.float32)], axis=0)
        acc_ref[...] += upd

        z = jnp.where(neg, s, _NEG_INF).reshape(gb * 24, 24)
        z_ref[...] = jnp.concatenate(
            [z, jnp.full((gb * 24, 104), _NEG_INF, jnp.float32)], axis=1)

    return pl.pallas_call(
        body,
        grid=(grid,),
        in_specs=[
            pl.BlockSpec((gb, 120, 24), lambda g: (g, 0, 0)),
            pl.BlockSpec((gb, 120, 24), lambda g: (g, 0, 0)),
            pl.BlockSpec((rv, 96), lambda g: (g, 0)),
            pl.BlockSpec((rv, 96), lambda g: (g, 0)),
        ],
        out_specs=[
            pl.BlockSpec((16, 128), lambda g: (0, 0)),
            pl.BlockSpec((gb * 24, 128), lambda g: (g, 0)),
        ],
        out_shape=[
            jax.ShapeDtypeStruct((16, 128), jnp.float32),
            jax.ShapeDtypeStruct((ngroups * 24, 128), jnp.float32),
        ],
    )(o3, l3, im2, re2)


def _sc_topk(z_flat):
    per_tile = z_flat.shape[0] // _NW
    rows = per_tile // 128

    mesh = plsc.VectorSubcoreMesh(
        core_axis_name="c", subcore_axis_name="s",
        num_cores=_NC, num_subcores=_NS)

    @functools.partial(
        pl.kernel,
        out_type=jax.ShapeDtypeStruct((_NW, 16), jnp.float32),
        mesh=mesh,
        compiler_params=pltpu.CompilerParams(needs_layout_passes=False),
        scratch_types=[
            pltpu.VMEM((per_tile,), jnp.float32),
            pltpu.VMEM((16,), jnp.float32),
        ],
    )
    def topk_kernel(z_hbm, cand_hbm, z_v, c_v):
        wid = lax.axis_index("s") * _NC + lax.axis_index("c")
        base = wid * per_tile
        pltpu.sync_copy(z_hbm.at[pl.ds(base, per_tile)], z_v)

        zero_idx = jnp.zeros((16,), jnp.int32)

        def merge(cand, z):
            zs = lax.sort(z)
            zrev = lax.rev(zs, (0,))
            return lax.sort(jnp.maximum(cand, zrev))

        def body(r, carry):
            cand, thresh = carry
            b = r * 128
            v0 = z_v[pl.ds(b, 16)]
            v1 = z_v[pl.ds(b + 16, 16)]
            zmax = jnp.maximum(v0, v1)
            hit = jnp.any(zmax > thresh)

            def do_merge():
                c = merge(merge(cand, v0), v1)
                return c, jnp.take(c, zero_idx)

            return lax.cond(hit, do_merge, lambda: (cand, thresh))

        init = (jnp.full((16,), _NEG_INF, jnp.float32),
                jnp.full((16,), _NEG_INF, jnp.float32))
        cand, _ = lax.fori_loop(0, rows, body, init)
        c_v[...] = cand
        pltpu.sync_copy(c_v, cand_hbm.at[wid])

    return topk_kernel(z_flat)


def _tc_finalize(acc, cands, k, n_vox):
    def body(acc_ref, cand_ref, out_ref):
        a = acc_ref[...]
        f_pos = jnp.sum(a[0:1, :])
        s1 = jnp.sum(a[1:2, :])
        s2 = jnp.sum(a[2:3, :])
        s3 = jnp.sum(a[3:4, :])
        s4 = jnp.sum(a[4:5, :])
        pos_cnt = jnp.sum(a[5:6, :])
        pos_cor = jnp.sum(a[6:7, :])
        neg_cnt = jnp.sum(a[7:8, :])
        sse = jnp.sum(a[8:9, :])

        z = cand_ref[...]
        zr, zc = z.shape
        ii = lax.broadcasted_iota(jnp.int32, (zr, zc), 0)
        ll = lax.broadcasted_iota(jnp.int32, (zr, zc), 1)
        fpos = ii * zc + ll
        neg_f = jnp.float32(0.0)
        neg_c = jnp.float32(0.0)
        for _ in range(k):
            m = jnp.max(z)
            p = _sigmoid(m)
            pt = 1.0 - p
            contrib = -((1.0 - pt) ** 2) * (_ALPHA * jnp.log(pt))
            valid = m > -jnp.inf
            neg_f += jnp.where(valid, contrib, 0.0)
            neg_c += jnp.where(valid & (p < 0.5), 1.0, 0.0)
            first = jnp.min(jnp.where(z == m, fpos, jnp.int32(2**30)))
            z = jnp.where(fpos == first, -jnp.inf, z)

        neg_k = jnp.minimum(neg_cnt, jnp.float32(k))
        classify = (f_pos + neg_f) / (pos_cnt + neg_k)
        denom = jnp.maximum(pos_cnt, 1.0)
        rl1 = jnp.where(pos_cnt > 0, s1 / denom, 0.0)
        rl2 = jnp.where(pos_cnt > 0, s2 / denom, 0.0)
        rl3 = jnp.where(pos_cnt > 0, s3 / denom, 0.0)
        rl4 = jnp.where(pos_cnt > 0, s4 / denom, 0.0)
        recon = _RECON_SCALE * (sse / jnp.float32(n_vox))
        loss = classify + rl1 + rl2 + rl3 + rl4 + recon

        lo = lax.broadcasted_iota(jnp.int32, (1, 128), 1)
        vec = jnp.where(lo == 0, loss,
              jnp.where(lo == 1, classify,
              jnp.where(lo == 2, rl1,
              jnp.where(lo == 3, rl2,
              jnp.where(lo == 4, rl3,
              jnp.where(lo == 5, rl4,
              jnp.where(lo == 6, pos_cor,
              jnp.where(lo == 7, pos_cnt,
              jnp.where(lo == 8, neg_c,
              jnp.where(lo == 9, neg_k,
              jnp.where(lo == 10, recon, 0.0)))))))))))
        out_ref[...] = vec

    return pl.pallas_call(
        body,
        out_shape=jax.ShapeDtypeStruct((1, 128), jnp.float32),
    )(acc, cands)


def kernel(output, labels, images, reconstructions):
    b, g1, g2, g3, na, nw = output.shape
    n_rows = output.size // nw
    k = min(_NUM_HARD * b, n_rows)

    ot = jnp.transpose(output, (0, 1, 4, 5, 2, 3)).reshape(-1, 120, g3)
    lt = jnp.transpose(labels, (0, 1, 4, 5, 2, 3)).reshape(-1, 120, g3)
    im2 = images.reshape(-1, images.shape[-1])
    re2 = reconstructions.reshape(-1, reconstructions.shape[-1])

    acc, z = _tc_main(ot, lt, im2, re2, grid=24)
    cands = _sc_topk(z.reshape(-1))
    res = _tc_finalize(acc, cands.reshape(4, 128), k, images.size)

    r = res[0]
    i32 = jnp.int32
    return (
        r[0], r[1], r[2], r[3], r[4], r[5],
        r[6].astype(i32), r[7].astype(i32),
        r[8].astype(i32), r[9].astype(i32),
        r[10],
    )


# speedup vs baseline: 41.6539x; 1.1460x over previous
import functools

import jax
import jax.numpy as jnp
from jax import lax
from jax.experimental import pallas as pl
from jax.experimental.pallas import tpu as pltpu
from jax.experimental.pallas import tpu_sc as plsc

_GAMMA = 2.0
_ALPHA = 0.5
_NUM_HARD = 2
_RECON_SCALE = 1e-06

_NC = 2
_NS = 16
_NW = _NC * _NS

_NEG_INF = float("-inf")


def _sigmoid(x):
    return jnp.where(
        x >= 0.0,
        1.0 / (1.0 + jnp.exp(-x)),
        jnp.exp(x) / (1.0 + jnp.exp(x)),
    )


def _tc_main(o3, l3, im2, re2, grid):
    ngroups = o3.shape[0]
    gb = ngroups // grid
    rv = im2.shape[0] // grid

    def body(o_ref, l_ref, im_ref, re_ref, acc_ref, z_ref):
        g = pl.program_id(0)

        @pl.when(g == 0)
        def _init():
            acc_ref[...] = jnp.zeros_like(acc_ref)

        s = o_ref[:, 0:24, :]
        c = l_ref[:, 0:24, :]
        pos = c > 0.5
        neg = c < -0.5
        p = _sigmoid(s)
        logp = jnp.log(p)
        focal = -((1.0 - p) ** 2) * ((1.0 - _ALPHA) * logp)
        focal = jnp.where(pos, focal, 0.0)

        def lanesum(x):
            v = jnp.sum(x.reshape(gb * 24, 24), axis=0, keepdims=True)
            return jnp.concatenate(
                [v, jnp.zeros((1, 104), jnp.float32)], axis=1)

        rows = [lanesum(focal)]
        for w in (1, 2, 3, 4):
            dw = o_ref[:, 24 * w:24 * w + 24, :] - l_ref[:, 24 * w:24 * w + 24, :]
            ad = jnp.abs(dw)
            l1 = jnp.where(ad < 1.0, 0.5 * dw * dw, ad - 0.5)
            rows.append(lanesum(jnp.where(pos, l1, 0.0)))
        rows.append(lanesum(pos.astype(jnp.float32)))
        rows.append(lanesum((pos & (p >= 0.5)).astype(jnp.float32)))
        rows.append(lanesum(neg.astype(jnp.float32)))

        dr = re_ref[...] - im_ref[...]
        v = jnp.sum(dr * dr, axis=0, keepdims=True)
        rows.append(jnp.concatenate(
            [v, jnp.zeros((1, 32), jnp.float32)], axis=1))

        upd = jnp.concatenate(rows, axis=0)
        upd = jnp.concatenate(
            [upd, jnp.zeros((16 - len(rows), 128), jnp.float32)], axis=0)
        acc_ref[...] += upd

        z = jnp.where(neg, s, _NEG_INF).reshape(gb * 24, 24)
        z_ref[...] = jnp.concatenate(
            [z, jnp.full((gb * 24, 104), _NEG_INF, jnp.float32)], axis=1)

    return pl.pallas_call(
        body,
        grid=(grid,),
        in_specs=[
            pl.BlockSpec((gb, 120, 24), lambda g: (g, 0, 0)),
            pl.BlockSpec((gb, 120, 24), lambda g: (g, 0, 0)),
            pl.BlockSpec((rv, 96), lambda g: (g, 0)),
            pl.BlockSpec((rv, 96), lambda g: (g, 0)),
        ],
        out_specs=[
            pl.BlockSpec((16, 128), lambda g: (0, 0)),
            pl.BlockSpec((gb * 24, 128), lambda g: (g, 0)),
        ],
        out_shape=[
            jax.ShapeDtypeStruct((16, 128), jnp.float32),
            jax.ShapeDtypeStruct((ngroups * 24, 128), jnp.float32),
        ],
    )(o3, l3, im2, re2)


def _sc_topk(z_flat):
    per_tile = z_flat.shape[0] // _NW
    rows = per_tile // 128

    mesh = plsc.VectorSubcoreMesh(
        core_axis_name="c", subcore_axis_name="s",
        num_cores=_NC, num_subcores=_NS)

    ch = 4
    nchunks = rows // ch
    cwords = ch * 128

    @functools.partial(
        pl.kernel,
        out_type=jax.ShapeDtypeStruct((_NW, 16), jnp.float32),
        mesh=mesh,
        compiler_params=pltpu.CompilerParams(needs_layout_passes=False),
        scratch_types=[
            pltpu.VMEM((per_tile,), jnp.float32),
            pltpu.VMEM((16,), jnp.float32),
        ],
    )
    def topk_kernel(z_hbm, cand_hbm, z_v, c_v):
        wid = lax.axis_index("s") * _NC + lax.axis_index("c")
        base = wid * per_tile
        pltpu.sync_copy(z_hbm.at[pl.ds(base, per_tile)], z_v)

        zero_idx = jnp.zeros((16,), jnp.int32)

        def vecs(b):
            return [z_v[pl.ds(b + r * 128 + o, 16)]
                    for r in range(ch) for o in (0, 16)]

        def p1(i, m):
            for v in vecs(i * cwords):
                m = jnp.maximum(m, v)
            return m

        m = lax.fori_loop(0, nchunks,
                          p1, jnp.full((16,), _NEG_INF, jnp.float32))
        cand0 = lax.sort(m)
        thresh0 = jnp.take(cand0, zero_idx)

        def merge(cand, z):
            zs = lax.sort(z)
            zrev = lax.rev(zs, (0,))
            return lax.sort(jnp.maximum(cand, zrev))

        def p2(i, carry):
            cand, thresh = carry
            vs = vecs(i * cwords)
            zmax = vs[0]
            for v in vs[1:]:
                zmax = jnp.maximum(zmax, v)
            hit = jnp.any(zmax > thresh)

            def do_merge():
                c, t = cand, thresh
                for v in vs:
                    def _yes(c=c, v=v):
                        cm = merge(c, v)
                        return cm, jnp.take(cm, zero_idx)

                    def _no(c=c, t=t):
                        return c, t

                    c, t = lax.cond(jnp.any(v > t), _yes, _no)
                return c, t

            return lax.cond(hit, do_merge, lambda: (cand, thresh))

        cand, _ = lax.fori_loop(0, nchunks, p2, (cand0, thresh0))
        c_v[...] = cand
        pltpu.sync_copy(c_v, cand_hbm.at[wid])

    return topk_kernel(z_flat)


def _tc_finalize(acc, cands_t, k, n_vox):
    def body(acc_ref, cand_ref, out_ref):
        a = acc_ref[...]
        f_pos = jnp.sum(a[0:1, :])
        s1 = jnp.sum(a[1:2, :])
        s2 = jnp.sum(a[2:3, :])
        s3 = jnp.sum(a[3:4, :])
        s4 = jnp.sum(a[4:5, :])
        pos_cnt = jnp.sum(a[5:6, :])
        pos_cor = jnp.sum(a[6:7, :])
        neg_cnt = jnp.sum(a[7:8, :])
        sse = jnp.sum(a[8:9, :])

        x = cand_ref[...]
        si = lax.broadcasted_iota(jnp.int32, (16, 128), 0)
        li = lax.broadcasted_iota(jnp.int32, (16, 128), 1)

        def swap_sub(v, d):
            return jnp.where((si & d) == 0,
                             pltpu.roll(v, 16 - d, axis=0),
                             pltpu.roll(v, d, axis=0))

        def swap_lane(v, d):
            return jnp.where((li & d) == 0,
                             pltpu.roll(v, 128 - d, axis=1),
                             pltpu.roll(v, d, axis=1))

        def rank_rev(v):
            for d in (8, 4, 2, 1):
                v = swap_sub(v, d)
            return v

        def rank_sort(v):
            for d in (8, 4, 2, 1):
                p = swap_sub(v, d)
                v = jnp.where((si & d) == 0,
                              jnp.minimum(v, p), jnp.maximum(v, p))
            return v

        for d in (1, 2, 4, 8, 16):
            x = rank_sort(jnp.maximum(x, rank_rev(swap_lane(x, d))))

        valid = (x > -jnp.inf) & (li == 0)
        p = _sigmoid(x)
        pt = 1.0 - p
        contrib = -((1.0 - pt) ** 2) * (_ALPHA * jnp.log(pt))
        neg_f = jnp.sum(jnp.where(valid, contrib, 0.0))
        neg_c = jnp.sum(jnp.where(valid & (p < 0.5), 1.0, 0.0))

        neg_k = jnp.minimum(neg_cnt, jnp.float32(k))
        classify = (f_pos + neg_f) / (pos_cnt + neg_k)
        denom = jnp.maximum(pos_cnt, 1.0)
        rl1 = jnp.where(pos_cnt > 0, s1 / denom, 0.0)
        rl2 = jnp.where(pos_cnt > 0, s2 / denom, 0.0)
        rl3 = jnp.where(pos_cnt > 0, s3 / denom, 0.0)
        rl4 = jnp.where(pos_cnt > 0, s4 / denom, 0.0)
        recon = _RECON_SCALE * (sse / jnp.float32(n_vox))
        loss = classify + rl1 + rl2 + rl3 + rl4 + recon

        lo = lax.broadcasted_iota(jnp.int32, (1, 128), 1)
        vec = jnp.where(lo == 0, loss,
              jnp.where(lo == 1, classify,
              jnp.where(lo == 2, rl1,
              jnp.where(lo == 3, rl2,
              jnp.where(lo == 4, rl3,
              jnp.where(lo == 5, rl4,
              jnp.where(lo == 6, pos_cor,
              jnp.where(lo == 7, pos_cnt,
              jnp.where(lo == 8, neg_c,
              jnp.where(lo == 9, neg_k,
              jnp.where(lo == 10, recon, 0.0)))))))))))
        out_ref[...] = vec

    return pl.pallas_call(
        body,
        out_shape=jax.ShapeDtypeStruct((1, 128), jnp.float32),
    )(acc, cands_t)


def kernel(output, labels, images, reconstructions):
    b, g1, g2, g3, na, nw = output.shape
    n_rows = output.size // nw
    k = min(_NUM_HARD * b, n_rows)

    ot = jnp.transpose(output, (0, 1, 4, 5, 2, 3)).reshape(-1, 120, g3)
    lt = jnp.transpose(labels, (0, 1, 4, 5, 2, 3)).reshape(-1, 120, g3)
    im2 = images.reshape(-1, images.shape[-1])
    re2 = reconstructions.reshape(-1, reconstructions.shape[-1])

    acc, z = _tc_main(ot, lt, im2, re2, grid=24)
    cands = _sc_topk(z.reshape(-1))
    cands_t = jnp.pad(cands.T, ((0, 0), (0, 128 - _NW)),
                      constant_values=_NEG_INF)
    res = _tc_finalize(acc, cands_t, k, images.size)

    r = res[0]
    i32 = jnp.int32
    return (
        r[0], r[1], r[2], r[3], r[4], r[5],
        r[6].astype(i32), r[7].astype(i32),
        r[8].astype(i32), r[9].astype(i32),
        r[10],
    )


# speedup vs baseline: 42.9984x; 1.0323x over previous
import functools

import jax
import jax.numpy as jnp
from jax import lax
from jax.experimental import pallas as pl
from jax.experimental.pallas import tpu as pltpu
from jax.experimental.pallas import tpu_sc as plsc

_GAMMA = 2.0
_ALPHA = 0.5
_NUM_HARD = 2
_RECON_SCALE = 1e-06

_NC = 2
_NS = 16
_NW = _NC * _NS

_NEG_INF = float("-inf")


def _sigmoid(x):
    return jnp.where(
        x >= 0.0,
        1.0 / (1.0 + jnp.exp(-x)),
        jnp.exp(x) / (1.0 + jnp.exp(x)),
    )


def _tc_main(o3, l3, im2, re2, grid):
    ngroups = o3.shape[0]
    gb = ngroups // grid
    rv = im2.shape[0] // grid

    def body(o_ref, l_ref, im_ref, re_ref, acc_ref, z_ref):
        g = pl.program_id(0)

        @pl.when(g == 0)
        def _init():
            acc_ref[...] = jnp.zeros_like(acc_ref)

        s = o_ref[:, 0:24, :]
        c = l_ref[:, 0:24, :]
        pos = c > 0.5
        neg = c < -0.5
        p = _sigmoid(s)
        logp = jnp.log(p)
        focal = -((1.0 - p) ** 2) * ((1.0 - _ALPHA) * logp)
        focal = jnp.where(pos, focal, 0.0)

        def lanesum(x):
            v = jnp.sum(x.reshape(gb * 24, 24), axis=0, keepdims=True)
            return jnp.concatenate(
                [v, jnp.zeros((1, 104), jnp.float32)], axis=1)

        rows = [lanesum(focal)]
        for w in (1, 2, 3, 4):
            dw = o_ref[:, 24 * w:24 * w + 24, :] - l_ref[:, 24 * w:24 * w + 24, :]
            ad = jnp.abs(dw)
            l1 = jnp.where(ad < 1.0, 0.5 * dw * dw, ad - 0.5)
            rows.append(lanesum(jnp.where(pos, l1, 0.0)))
        rows.append(lanesum(pos.astype(jnp.float32)))
        rows.append(lanesum((pos & (p >= 0.5)).astype(jnp.float32)))
        rows.append(lanesum(neg.astype(jnp.float32)))

        dr = re_ref[...] - im_ref[...]
        v = jnp.sum(dr * dr, axis=0, keepdims=True)
        rows.append(jnp.concatenate(
            [v, jnp.zeros((1, 32), jnp.float32)], axis=1))

        upd = jnp.concatenate(rows, axis=0)
        upd = jnp.concatenate(
            [upd, jnp.zeros((16 - len(rows), 128), jnp.float32)], axis=0)
        acc_ref[...] += upd

        z = jnp.where(neg, s, _NEG_INF).reshape(gb * 24, 24)
        z_ref[...] = jnp.concatenate(
            [z, jnp.full((gb * 24, 104), _NEG_INF, jnp.float32)], axis=1)

    return pl.pallas_call(
        body,
        grid=(grid,),
        in_specs=[
            pl.BlockSpec((gb, 120, 24), lambda g: (g, 0, 0)),
            pl.BlockSpec((gb, 120, 24), lambda g: (g, 0, 0)),
            pl.BlockSpec((rv, 96), lambda g: (g, 0)),
            pl.BlockSpec((rv, 96), lambda g: (g, 0)),
        ],
        out_specs=[
            pl.BlockSpec((16, 128), lambda g: (0, 0)),
            pl.BlockSpec((gb * 24, 128), lambda g: (g, 0)),
        ],
        out_shape=[
            jax.ShapeDtypeStruct((16, 128), jnp.float32),
            jax.ShapeDtypeStruct((ngroups * 24, 128), jnp.float32),
        ],
    )(o3, l3, im2, re2)


def _sc_topk(z_flat):
    per_tile = z_flat.shape[0] // _NW
    rows = per_tile // 128

    mesh = plsc.VectorSubcoreMesh(
        core_axis_name="c", subcore_axis_name="s",
        num_cores=_NC, num_subcores=_NS)

    ch = 4
    nchunks = rows // ch
    cwords = ch * 128

    @functools.partial(
        pl.kernel,
        out_type=jax.ShapeDtypeStruct((_NW, 16), jnp.float32),
        mesh=mesh,
        compiler_params=pltpu.CompilerParams(needs_layout_passes=False),
        scratch_types=[
            pltpu.VMEM((per_tile,), jnp.float32),
            pltpu.VMEM((16,), jnp.float32),
        ],
    )
    def topk_kernel(z_hbm, cand_hbm, z_v, c_v):
        wid = lax.axis_index("s") * _NC + lax.axis_index("c")
        base = wid * per_tile
        pltpu.sync_copy(z_hbm.at[pl.ds(base, per_tile)], z_v)

        zero_idx = jnp.zeros((16,), jnp.int32)

        def vecs(b):
            return [z_v[pl.ds(b + r * 128 + o, 16)]
                    for r in range(ch) for o in (0, 16)]

        def p1(i, m):
            for v in vecs(i * cwords):
                m = jnp.maximum(m, v)
            return m

        m = lax.fori_loop(0, nchunks,
                          p1, jnp.full((16,), _NEG_INF, jnp.float32))
        cand0 = lax.sort(m)
        thresh0 = jnp.take(cand0, zero_idx)

        def merge(cand, z):
            zs = lax.sort(z)
            zrev = lax.rev(zs, (0,))
            return lax.sort(jnp.maximum(cand, zrev))

        def p2(i, carry):
            cand, thresh = carry
            vs = vecs(i * cwords)
            zmax = vs[0]
            for v in vs[1:]:
                zmax = jnp.maximum(zmax, v)
            hit = jnp.any(zmax > thresh)

            def do_merge():
                c, t = cand, thresh
                for v in vs:
                    def _yes(c=c, v=v):
                        cm = merge(c, v)
                        return cm, jnp.take(cm, zero_idx)

                    def _no(c=c, t=t):
                        return c, t

                    c, t = lax.cond(jnp.any(v > t), _yes, _no)
                return c, t

            return lax.cond(hit, do_merge, lambda: (cand, thresh))

        cand, _ = lax.fori_loop(0, nchunks, p2, (cand0, thresh0))
        c_v[...] = cand
        pltpu.sync_copy(c_v, cand_hbm.at[wid])

    return topk_kernel(z_flat)


def _tc_finalize(acc, cands_t, k, n_vox):
    def body(acc_ref, cand_ref, out_ref):
        a = acc_ref[...]
        f_pos = jnp.sum(a[0:1, :])
        s1 = jnp.sum(a[1:2, :])
        s2 = jnp.sum(a[2:3, :])
        s3 = jnp.sum(a[3:4, :])
        s4 = jnp.sum(a[4:5, :])
        pos_cnt = jnp.sum(a[5:6, :])
        pos_cor = jnp.sum(a[6:7, :])
        neg_cnt = jnp.sum(a[7:8, :])
        sse = jnp.sum(a[8:9, :])

        x = cand_ref[...]
        si = lax.broadcasted_iota(jnp.int32, (16, 128), 0)
        li = lax.broadcasted_iota(jnp.int32, (16, 128), 1)

        def swap_sub(v, d):
            return jnp.where((si & d) == 0,
                             pltpu.roll(v, 16 - d, axis=0),
                             pltpu.roll(v, d, axis=0))

        def swap_lane(v, d):
            return jnp.where((li & d) == 0,
                             pltpu.roll(v, 128 - d, axis=1),
                             pltpu.roll(v, d, axis=1))

        def rank_rev(v):
            for d in (8, 4, 2, 1):
                v = swap_sub(v, d)
            return v

        def rank_sort(v):
            for d in (8, 4, 2, 1):
                p = swap_sub(v, d)
                v = jnp.where((si & d) == 0,
                              jnp.minimum(v, p), jnp.maximum(v, p))
            return v

        for d in (1, 2, 4, 8, 16):
            x = rank_sort(jnp.maximum(x, rank_rev(swap_lane(x, d))))

        valid = (x > -jnp.inf) & (li == 0)
        p = _sigmoid(x)
        pt = 1.0 - p
        contrib = -((1.0 - pt) ** 2) * (_ALPHA * jnp.log(pt))
        neg_f = jnp.sum(jnp.where(valid, contrib, 0.0))
        neg_c = jnp.sum(jnp.where(valid & (p < 0.5), 1.0, 0.0))

        neg_k = jnp.minimum(neg_cnt, jnp.float32(k))
        classify = (f_pos + neg_f) / (pos_cnt + neg_k)
        denom = jnp.maximum(pos_cnt, 1.0)
        rl1 = jnp.where(pos_cnt > 0, s1 / denom, 0.0)
        rl2 = jnp.where(pos_cnt > 0, s2 / denom, 0.0)
        rl3 = jnp.where(pos_cnt > 0, s3 / denom, 0.0)
        rl4 = jnp.where(pos_cnt > 0, s4 / denom, 0.0)
        recon = _RECON_SCALE * (sse / jnp.float32(n_vox))
        loss = classify + rl1 + rl2 + rl3 + rl4 + recon

        lo = lax.broadcasted_iota(jnp.int32, (1, 128), 1)
        vec = jnp.where(lo == 0, loss,
              jnp.where(lo == 1, classify,
              jnp.where(lo == 2, rl1,
              jnp.where(lo == 3, rl2,
              jnp.where(lo == 4, rl3,
              jnp.where(lo == 5, rl4,
              jnp.where(lo == 6, pos_cor,
              jnp.where(lo == 7, pos_cnt,
              jnp.where(lo == 8, neg_c,
              jnp.where(lo == 9, neg_k,
              jnp.where(lo == 10, recon, 0.0)))))))))))
        out_ref[...] = vec

    return pl.pallas_call(
        body,
        out_shape=jax.ShapeDtypeStruct((1, 128), jnp.float32),
    )(acc, cands_t)


def kernel(output, labels, images, reconstructions):
    b, g1, g2, g3, na, nw = output.shape
    n_rows = output.size // nw
    k = min(_NUM_HARD * b, n_rows)

    ot = jnp.transpose(output, (0, 1, 4, 5, 2, 3)).reshape(-1, 120, g3)
    lt = jnp.transpose(labels, (0, 1, 4, 5, 2, 3)).reshape(-1, 120, g3)
    im2 = images.reshape(-1, images.shape[-1])
    re2 = reconstructions.reshape(-1, reconstructions.shape[-1])

    acc, z = _tc_main(ot, lt, im2, re2, grid=12)
    cands = _sc_topk(z.reshape(-1))
    cands_t = jnp.pad(cands.T, ((0, 0), (0, 128 - _NW)),
                      constant_values=_NEG_INF)
    res = _tc_finalize(acc, cands_t, k, images.size)

    r = res[0]
    i32 = jnp.int32
    return (
        r[0], r[1], r[2], r[3], r[4], r[5],
        r[6].astype(i32), r[7].astype(i32),
        r[8].astype(i32), r[9].astype(i32),
        r[10],
    )
